# baseline (speedup 1.0000x reference)
import jax
import jax.numpy as jnp
from jax import lax
from jax.experimental import pallas as pl
from jax.experimental.pallas import tpu as pltpu

_CB = 2
_NBUF = 4
_PF = 2


def _in_copy(patch_hbm, bufs, in_sems, i, k):
    return pltpu.make_async_copy(
        patch_hbm.at[pl.ds(i * _CB, _CB)], bufs.at[k], in_sems.at[k]
    )


def _out_copy(out_hbm, bufs, out_sems, i, k):
    return pltpu.make_async_copy(
        bufs.at[k], out_hbm.at[pl.ds(i * _CB, _CB)], out_sems.at[k]
    )


def _pipe_kernel(nch, patch_hbm, pos_ref, out_hbm, bufs, in_sems, out_sems):
    pos = pos_ref[...][None]

    def step(i, k, *, prefetch=True, wait_out=True):
        _in_copy(patch_hbm, bufs, in_sems, i, k).wait()
        bufs[k] = bufs[k] + pos
        _out_copy(out_hbm, bufs, out_sems, i, k).start()
        if prefetch:
            j = i + _PF
            kj = (k + _PF) % _NBUF
            if wait_out:
                _out_copy(out_hbm, bufs, out_sems, j - _NBUF, kj).wait()
            _in_copy(patch_hbm, bufs, in_sems, j, kj).start()

    for k in range(_PF):
        _in_copy(patch_hbm, bufs, in_sems, k, k).start()

    for k in range(_NBUF):
        step(k, k, wait_out=(k + _PF >= _NBUF))

    def body(g, carry):
        base = g * _NBUF
        for k in range(_NBUF):
            step(base + k, k)
        return carry

    lax.fori_loop(1, nch // _NBUF - 1, body, 0)

    base = nch - _NBUF
    for k in range(_NBUF):
        step(base + k, k, prefetch=(k + _PF < _NBUF))

    for k in range(_NBUF):
        _out_copy(out_hbm, bufs, out_sems, nch - _NBUF + k, k).wait()


def kernel(patch, pos_table):
    B, N, D = patch.shape
    nch = B // _CB
    import functools
    return pl.pallas_call(
        functools.partial(_pipe_kernel, nch),
        in_specs=[
            pl.BlockSpec(memory_space=pl.ANY),
            pl.BlockSpec(memory_space=pltpu.VMEM),
        ],
        out_specs=pl.BlockSpec(memory_space=pl.ANY),
        out_shape=jax.ShapeDtypeStruct((B, N, D), patch.dtype),
        scratch_shapes=[
            pltpu.VMEM((_NBUF, _CB, N, D), patch.dtype),
            pltpu.SemaphoreType.DMA((_NBUF,)),
            pltpu.SemaphoreType.DMA((_NBUF,)),
        ],
    )(patch, pos_table)


# speedup vs baseline: 1.0255x; 1.0255x over previous
import jax
import jax.numpy as jnp
from jax import lax
from jax.experimental import pallas as pl
from jax.experimental.pallas import tpu as pltpu

_CB = 2
_NBUF = 8
_PF = 3


def _in_copy(patch_hbm, bufs, in_sems, i, k):
    return pltpu.make_async_copy(
        patch_hbm.at[pl.ds(i * _CB, _CB)], bufs.at[k], in_sems.at[k]
    )


def _out_copy(out_hbm, bufs, out_sems, i, k):
    return pltpu.make_async_copy(
        bufs.at[k], out_hbm.at[pl.ds(i * _CB, _CB)], out_sems.at[k]
    )


def _pipe_kernel(nch, patch_hbm, pos_ref, out_hbm, bufs, in_sems, out_sems):
    pos = pos_ref[...][None]

    def step(i, k, *, prefetch=True, wait_out=True):
        _in_copy(patch_hbm, bufs, in_sems, i, k).wait()
        bufs[k] = bufs[k] + pos
        _out_copy(out_hbm, bufs, out_sems, i, k).start()
        if prefetch:
            j = i + _PF
            kj = (k + _PF) % _NBUF
            if wait_out:
                _out_copy(out_hbm, bufs, out_sems, j - _NBUF, kj).wait()
            _in_copy(patch_hbm, bufs, in_sems, j, kj).start()

    for k in range(_PF):
        _in_copy(patch_hbm, bufs, in_sems, k, k).start()

    for k in range(_NBUF):
        step(k, k, wait_out=(k + _PF >= _NBUF))

    def body(g, carry):
        base = g * _NBUF
        for k in range(_NBUF):
            step(base + k, k)
        return carry

    lax.fori_loop(1, nch // _NBUF - 1, body, 0)

    base = nch - _NBUF
    for k in range(_NBUF):
        step(base + k, k, prefetch=(k + _PF < _NBUF))

    for k in range(_NBUF):
        _out_copy(out_hbm, bufs, out_sems, nch - _NBUF + k, k).wait()


def kernel(patch, pos_table):
    B, N, D = patch.shape
    nch = B // _CB
    import functools
    return pl.pallas_call(
        functools.partial(_pipe_kernel, nch),
        in_specs=[
            pl.BlockSpec(memory_space=pl.ANY),
            pl.BlockSpec(memory_space=pltpu.VMEM),
        ],
        out_specs=pl.BlockSpec(memory_space=pl.ANY),
        out_shape=jax.ShapeDtypeStruct((B, N, D), patch.dtype),
        scratch_shapes=[
            pltpu.VMEM((_NBUF, _CB, N, D), patch.dtype),
            pltpu.SemaphoreType.DMA((_NBUF,)),
            pltpu.SemaphoreType.DMA((_NBUF,)),
        ],
    )(patch, pos_table)
